# baseline (speedup 1.0000x reference)
import math

import jax
import jax.numpy as jnp
from jax.experimental import pallas as pl
from jax.experimental.pallas import tpu as pltpu

D = 1024
H = 16
T = 2048
BLK = 64
NB = T // BLK
KC = int(math.ceil(0.5 * NB))
DH = D // H
SCALE = 1.0 / math.sqrt(DH)
ROWS = 256
QT = 1024
GB = QT // BLK

_BF = jnp.bfloat16


HG = 4


def _qkv_sel_kernel(xn_ref, wq_ref, bq_ref, wk_ref, bk_ref,
                    wv_ref, bv_ref, q_ref, k_ref, v_ref, mask_ref):
    xn = xn_ref[...]
    w = jnp.concatenate([wq_ref[...].reshape(HG * DH, D),
                         wk_ref[...].reshape(HG * DH, D),
                         wv_ref[...].reshape(HG * DH, D)],
                        axis=0).astype(_BF)
    qkv = jax.lax.dot_general(xn, w, (((1,), (1,)), ((), ())),
                              preferred_element_type=jnp.float32)
    for j in range(HG):
        qh = qkv[:, j * DH:(j + 1) * DH] + bq_ref[0, :, j * DH:(j + 1) * DH]
        kh = (qkv[:, (HG + j) * DH:(HG + j + 1) * DH]
              + bk_ref[0, :, j * DH:(j + 1) * DH])
        vh = (qkv[:, (2 * HG + j) * DH:(2 * HG + j + 1) * DH]
              + bv_ref[0, :, j * DH:(j + 1) * DH])
        q_ref[j] = qh
        k_ref[j] = kh
        v_ref[j] = vh
        qm = jnp.mean(qh.reshape(NB, BLK, DH), axis=1)
        km = jnp.mean(kh.reshape(NB, BLK, DH), axis=1)
        sim = jax.lax.dot_general(qm.astype(_BF), km.astype(_BF),
                                  (((1,), (1,)), ((), ())),
                                  preferred_element_type=jnp.float32)
        gt = (sim[:, None, :] > sim[:, :, None]).astype(jnp.float32)
        cntg = jnp.sum(gt, axis=-1)
        ii = jax.lax.broadcasted_iota(jnp.int32, (NB, NB), 0)
        jj = jax.lax.broadcasted_iota(jnp.int32, (NB, NB), 1)
        keep = (cntg < float(KC)) | (jj == ii)
        drop = 1.0 - keep.astype(jnp.float32)
        j_io = jax.lax.broadcasted_iota(jnp.int32, (NB, T), 0)
        c_io = jax.lax.broadcasted_iota(jnp.int32, (NB, T), 1)
        expander = (c_io // BLK == j_io).astype(_BF)
        mask_ref[j] = jax.lax.dot_general(
            (drop * (-1e30)).astype(_BF), expander, (((1,), (0,)), ((), ())),
            preferred_element_type=jnp.float32)


def _attn_kernel(q_ref, k_ref, v_ref, mask_ref, o_ref):
    i = pl.program_id(1)
    q = (q_ref[0] * SCALE).astype(_BF)

    def _tile(g, causal_add):
        kb = k_ref[0, pl.ds(g * QT, QT), :].astype(_BF)
        vb = v_ref[0, pl.ds(g * QT, QT), :].astype(_BF)
        sc = jax.lax.dot_general(q, kb, (((1,), (1,)), ((), ())),
                                 preferred_element_type=jnp.float32)
        m4 = mask_ref[0, 0, :, pl.ds(g * QT, QT)]
        sc = (sc.reshape(GB, BLK, QT) + m4[:, None, :]).reshape(QT, QT)
        if causal_add is not None:
            sc = sc + causal_add
        return sc, vb

    def _update(sc, vb, m, l, acc):
        mnew = jnp.maximum(m, jnp.max(sc, axis=1, keepdims=True))
        alpha = jnp.exp(m - mnew)
        p = jnp.exp(sc - mnew)
        l2 = l * alpha + jnp.sum(p, axis=1, keepdims=True)
        acc2 = acc * alpha + jax.lax.dot_general(
            p.astype(_BF), vb, (((1,), (0,)), ((), ())),
            preferred_element_type=jnp.float32)
        return mnew, l2, acc2

    def body(g, carry):
        m, l, acc = carry
        sc, vb = _tile(g, None)
        return _update(sc, vb, m, l, acc)

    m0 = jnp.full((QT, 1), -1e30, jnp.float32)
    l0 = jnp.zeros((QT, 1), jnp.float32)
    a0 = jnp.zeros((QT, DH), jnp.float32)
    m, l, acc = jax.lax.fori_loop(0, i, body, (m0, l0, a0))
    HQ = QT // 2
    row = jax.lax.broadcasted_iota(jnp.int32, (QT, HQ), 0)
    col = jax.lax.broadcasted_iota(jnp.int32, (QT, HQ), 1)
    base = i * QT

    kb = k_ref[0, pl.ds(base, HQ), :].astype(_BF)
    vb = v_ref[0, pl.ds(base, HQ), :].astype(_BF)
    sc = jax.lax.dot_general(q, kb, (((1,), (1,)), ((), ())),
                             preferred_element_type=jnp.float32)
    m4 = mask_ref[0, 0, :, pl.ds(base, HQ)]
    sc = (sc.reshape(GB, BLK, HQ) + m4[:, None, :]).reshape(QT, HQ)
    sc = sc + jnp.where(row >= col, 0.0, -1e30)
    m, l, acc = _update(sc, vb, m, l, acc)

    kb = k_ref[0, pl.ds(base + HQ, HQ), :].astype(_BF)
    vb = v_ref[0, pl.ds(base + HQ, HQ), :].astype(_BF)
    q2 = q[HQ:, :]
    sc = jax.lax.dot_general(q2, kb, (((1,), (1,)), ((), ())),
                             preferred_element_type=jnp.float32)
    m4 = mask_ref[0, 0, GB // 2:, pl.ds(base + HQ, HQ)]
    sc = (sc.reshape(GB // 2, BLK, HQ) + m4[:, None, :]).reshape(HQ, HQ)
    sc = sc + jnp.where(row[HQ:, :] - HQ >= col[HQ:, :], 0.0, -1e30)
    mh, lh, acch = _update(sc, vb, m[HQ:], l[HQ:], acc[HQ:])
    m = jnp.concatenate([m[:HQ], mh], axis=0)
    l = jnp.concatenate([l[:HQ], lh], axis=0)
    acc = jnp.concatenate([acc[:HQ], acch], axis=0)
    o_ref[0] = acc / l


def _outproj_kernel(c_ref, wo_ref, bo_ref, o_ref):
    o_ref[...] = jax.lax.dot_general(
        c_ref[...].astype(_BF), wo_ref[...].astype(_BF),
        (((1,), (1,)), ((), ())),
        preferred_element_type=jnp.float32) + bo_ref[...]


def kernel(inputs, ln_g, ln_b, Wq, bq, Wk, bk, Wv, bv, Wo, bo):
    x = inputs.reshape(T, D)
    mu = jnp.mean(x, axis=-1, keepdims=True)
    var = jnp.mean((x - mu) ** 2, axis=-1, keepdims=True)
    xn = ((x - mu) / jnp.sqrt(var + 1e-5) * ln_g.reshape(1, D)
          + ln_b.reshape(1, D)).astype(_BF)
    wq3 = Wq.reshape(H, DH, D)
    wk3 = Wk.reshape(H, DH, D)
    wv3 = Wv.reshape(H, DH, D)
    bq3 = bq.reshape(H // HG, 1, HG * DH)
    bk3 = bk.reshape(H // HG, 1, HG * DH)
    bv3 = bv.reshape(H // HG, 1, HG * DH)
    bo2 = bo.reshape(1, D)

    fullx = pl.BlockSpec((T, D), lambda h: (0, 0))
    whead = pl.BlockSpec((HG, DH, D), lambda h: (h, 0, 0))
    bhead = pl.BlockSpec((1, 1, HG * DH), lambda h: (h, 0, 0))
    ohead = pl.BlockSpec((HG, T, DH), lambda h: (h, 0, 0))
    qh, kh, vh, amask = pl.pallas_call(
        _qkv_sel_kernel,
        grid=(H // HG,),
        in_specs=[fullx, whead, bhead, whead, bhead, whead, bhead],
        out_specs=[ohead, ohead, ohead,
                   pl.BlockSpec((HG, NB, T), lambda h: (h, 0, 0))],
        out_shape=[jax.ShapeDtypeStruct((H, T, DH), jnp.float32)] * 3 +
                  [jax.ShapeDtypeStruct((H, NB, T), jnp.float32)],
    )(xn, wq3, bq3, wk3, bk3, wv3, bv3)
    amask4 = amask.reshape(H, NB // GB, GB, T)

    ctx = pl.pallas_call(
        _attn_kernel,
        grid=(H, T // QT),
        in_specs=[
            pl.BlockSpec((1, QT, DH), lambda h, i: (h, i, 0)),
            pl.BlockSpec((1, T, DH), lambda h, i: (h, 0, 0)),
            pl.BlockSpec((1, T, DH), lambda h, i: (h, 0, 0)),
            pl.BlockSpec((1, 1, GB, T), lambda h, i: (h, i, 0, 0)),
        ],
        out_specs=pl.BlockSpec((1, QT, DH), lambda h, i: (h, i, 0)),
        out_shape=jax.ShapeDtypeStruct((H, T, DH), jnp.float32),
    )(qh, kh, vh, amask4)
    ctx2 = ctx.transpose(1, 0, 2).reshape(T, D)

    rows = pl.BlockSpec((ROWS, D), lambda r: (r, 0))
    out = pl.pallas_call(
        _outproj_kernel,
        grid=(T // ROWS,),
        in_specs=[rows, pl.BlockSpec((D, D), lambda r: (0, 0)),
                  pl.BlockSpec((1, D), lambda r: (0, 0))],
        out_specs=rows,
        out_shape=jax.ShapeDtypeStruct((T, D), jnp.float32),
    )(ctx2, Wo, bo2)

    return out.reshape(1, T, D)


# speedup vs baseline: 1.0950x; 1.0950x over previous
import math

import jax
import jax.numpy as jnp
from jax.experimental import pallas as pl
from jax.experimental.pallas import tpu as pltpu

D = 1024
H = 16
T = 2048
BLK = 64
NB = T // BLK
KC = int(math.ceil(0.5 * NB))
DH = D // H
SCALE = 1.0 / math.sqrt(DH)
ROWS = 256
QT = 1024
GB = QT // BLK

_BF = jnp.bfloat16


HG = 4


def _qkv_sel_kernel(xn_ref, wq_ref, bq_ref, wk_ref, bk_ref,
                    wv_ref, bv_ref, q_ref, k_ref, v_ref, mask_ref):
    xn = xn_ref[...]
    w = jnp.concatenate([wq_ref[...].reshape(HG * DH, D),
                         wk_ref[...].reshape(HG * DH, D),
                         wv_ref[...].reshape(HG * DH, D)],
                        axis=0).astype(_BF)
    qkv = jax.lax.dot_general(xn, w, (((1,), (1,)), ((), ())),
                              preferred_element_type=jnp.float32)
    for j in range(HG):
        qh = qkv[:, j * DH:(j + 1) * DH] + bq_ref[0, :, j * DH:(j + 1) * DH]
        kh = (qkv[:, (HG + j) * DH:(HG + j + 1) * DH]
              + bk_ref[0, :, j * DH:(j + 1) * DH])
        vh = (qkv[:, (2 * HG + j) * DH:(2 * HG + j + 1) * DH]
              + bv_ref[0, :, j * DH:(j + 1) * DH])
        q_ref[j] = qh
        k_ref[j] = kh
        v_ref[j] = vh
        qm = jnp.mean(qh.reshape(NB, BLK, DH), axis=1)
        km = jnp.mean(kh.reshape(NB, BLK, DH), axis=1)
        sim = jax.lax.dot_general(qm.astype(_BF), km.astype(_BF),
                                  (((1,), (1,)), ((), ())),
                                  preferred_element_type=jnp.float32)
        gt = (sim[:, None, :] > sim[:, :, None]).astype(jnp.float32)
        cntg = jnp.sum(gt, axis=-1)
        ii = jax.lax.broadcasted_iota(jnp.int32, (NB, NB), 0)
        jj = jax.lax.broadcasted_iota(jnp.int32, (NB, NB), 1)
        keep = (cntg < float(KC)) | (jj == ii)
        drop = 1.0 - keep.astype(jnp.float32)
        j_io = jax.lax.broadcasted_iota(jnp.int32, (NB, T), 0)
        c_io = jax.lax.broadcasted_iota(jnp.int32, (NB, T), 1)
        expander = (c_io // BLK == j_io).astype(_BF)
        mask_ref[j] = jax.lax.dot_general(
            (drop * (-1e30)).astype(_BF), expander, (((1,), (0,)), ((), ())),
            preferred_element_type=jnp.float32)


def _attn_kernel(q_ref, k_ref, v_ref, mask_ref, o_ref):
    i = pl.program_id(1)
    q = (q_ref[0] * SCALE).astype(_BF)

    def _tile(g, causal_add):
        kb = k_ref[0, pl.ds(g * QT, QT), :].astype(_BF)
        vb = v_ref[0, pl.ds(g * QT, QT), :].astype(_BF)
        sc = jax.lax.dot_general(q, kb, (((1,), (1,)), ((), ())),
                                 preferred_element_type=jnp.float32)
        m4 = mask_ref[0, 0, :, pl.ds(g * QT, QT)]
        sc = (sc.reshape(GB, BLK, QT) + m4[:, None, :]).reshape(QT, QT)
        if causal_add is not None:
            sc = sc + causal_add
        return sc, vb

    def _update(sc, vb, m, l, acc):
        mnew = jnp.maximum(m, jnp.max(sc, axis=1, keepdims=True))
        alpha = jnp.exp(m - mnew)
        p = jnp.exp(sc - mnew)
        l2 = l * alpha + jnp.sum(p, axis=1, keepdims=True)
        acc2 = acc * alpha + jax.lax.dot_general(
            p.astype(_BF), vb, (((1,), (0,)), ((), ())),
            preferred_element_type=jnp.float32)
        return mnew, l2, acc2

    def body(g, carry):
        m, l, acc = carry
        sc, vb = _tile(g, None)
        return _update(sc, vb, m, l, acc)

    m0 = jnp.full((QT, 1), -1e30, jnp.float32)
    l0 = jnp.zeros((QT, 1), jnp.float32)
    a0 = jnp.zeros((QT, DH), jnp.float32)
    m, l, acc = jax.lax.fori_loop(0, i, body, (m0, l0, a0))
    row = jax.lax.broadcasted_iota(jnp.int32, (QT, QT), 0)
    col = jax.lax.broadcasted_iota(jnp.int32, (QT, QT), 1)
    causal_add = jnp.where(row >= col, 0.0, -1e30)
    sc, vb = _tile(i, causal_add)
    m, l, acc = _update(sc, vb, m, l, acc)
    o_ref[0] = acc / l


def _outproj_kernel(c_ref, wo_ref, bo_ref, o_ref):
    o_ref[...] = jax.lax.dot_general(
        c_ref[...].astype(_BF), wo_ref[...].astype(_BF),
        (((1,), (1,)), ((), ())),
        preferred_element_type=jnp.float32) + bo_ref[...]


def kernel(inputs, ln_g, ln_b, Wq, bq, Wk, bk, Wv, bv, Wo, bo):
    x = inputs.reshape(T, D)
    mu = jnp.mean(x, axis=-1, keepdims=True)
    var = jnp.mean((x - mu) ** 2, axis=-1, keepdims=True)
    xn = ((x - mu) / jnp.sqrt(var + 1e-5) * ln_g.reshape(1, D)
          + ln_b.reshape(1, D)).astype(_BF)
    wq3 = Wq.reshape(H, DH, D)
    wk3 = Wk.reshape(H, DH, D)
    wv3 = Wv.reshape(H, DH, D)
    bq3 = bq.reshape(H // HG, 1, HG * DH)
    bk3 = bk.reshape(H // HG, 1, HG * DH)
    bv3 = bv.reshape(H // HG, 1, HG * DH)
    bo2 = bo.reshape(1, D)

    fullx = pl.BlockSpec((T, D), lambda h: (0, 0))
    whead = pl.BlockSpec((HG, DH, D), lambda h: (h, 0, 0))
    bhead = pl.BlockSpec((1, 1, HG * DH), lambda h: (h, 0, 0))
    ohead = pl.BlockSpec((HG, T, DH), lambda h: (h, 0, 0))
    qh, kh, vh, amask = pl.pallas_call(
        _qkv_sel_kernel,
        grid=(H // HG,),
        in_specs=[fullx, whead, bhead, whead, bhead, whead, bhead],
        out_specs=[ohead, ohead, ohead,
                   pl.BlockSpec((HG, NB, T), lambda h: (h, 0, 0))],
        out_shape=[jax.ShapeDtypeStruct((H, T, DH), jnp.float32)] * 3 +
                  [jax.ShapeDtypeStruct((H, NB, T), jnp.float32)],
    )(xn, wq3, bq3, wk3, bk3, wv3, bv3)
    amask4 = amask.reshape(H, NB // GB, GB, T)

    ctx = pl.pallas_call(
        _attn_kernel,
        grid=(H, T // QT),
        in_specs=[
            pl.BlockSpec((1, QT, DH), lambda h, i: (h, i, 0)),
            pl.BlockSpec((1, T, DH), lambda h, i: (h, 0, 0)),
            pl.BlockSpec((1, T, DH), lambda h, i: (h, 0, 0)),
            pl.BlockSpec((1, 1, GB, T), lambda h, i: (h, i, 0, 0)),
        ],
        out_specs=pl.BlockSpec((1, QT, DH), lambda h, i: (h, i, 0)),
        out_shape=jax.ShapeDtypeStruct((H, T, DH), jnp.float32),
    )(qh, kh, vh, amask4)
    ctx2 = ctx.transpose(1, 0, 2).reshape(T, D)

    rows = pl.BlockSpec((ROWS, D), lambda r: (r, 0))
    out = pl.pallas_call(
        _outproj_kernel,
        grid=(T // ROWS,),
        in_specs=[rows, pl.BlockSpec((D, D), lambda r: (0, 0)),
                  pl.BlockSpec((1, D), lambda r: (0, 0))],
        out_specs=rows,
        out_shape=jax.ShapeDtypeStruct((T, D), jnp.float32),
    )(ctx2, Wo, bo2)

    return out.reshape(1, T, D)
